# baseline (speedup 1.0000x reference)
import jax
import jax.numpy as jnp
from jax import lax
from jax.experimental import pallas as pl
from jax.experimental.pallas import tpu as pltpu
from jax.experimental.pallas import tpu_sc as plsc

N = 10000
E = 160000
D = 128
ED = 16
H = 4
C = 64
T = 8
GINE = 64
OUT = 128

NC = 2
NS = 16
NW = NC * NS
EK = 128
NCHUNK = E // EK
MAXCH = -(-NCHUNK // NW)
NP = 10240
RPT = NP // NS
ZCHUNKS = ((0, 128), (128, 128), (256, 128), (384, 128), (512, 128))

_BN = 1000
_BE = 2000

def _mesh():
    return plsc.VectorSubcoreMesh(
        core_axis_name="c", subcore_axis_name="s", num_cores=NC, num_subcores=NS)


def _tca_body(x_ref, wg_ref, asdw_ref, h0_ref, h1_ref, asd_ref):
    h = jnp.dot(x_ref[...], wg_ref[...], preferred_element_type=jnp.float32)
    h0_ref[...] = h[:, :128]
    h1_ref[...] = h[:, 128:]
    asd_ref[...] = jnp.dot(h, asdw_ref[...], preferred_element_type=jnp.float32)


def _tca(x, W_gat, Asd):
    return pl.pallas_call(
        _tca_body,
        grid=(N // _BN,),
        in_specs=[
            pl.BlockSpec((_BN, D), lambda i: (i, 0)),
            pl.BlockSpec((D, H * C), lambda i: (0, 0)),
            pl.BlockSpec((H * C, 8), lambda i: (0, 0)),
        ],
        out_specs=[
            pl.BlockSpec((_BN, 128), lambda i: (i, 0)),
            pl.BlockSpec((_BN, 128), lambda i: (i, 0)),
            pl.BlockSpec((_BN, 8), lambda i: (i, 0)),
        ],
        out_shape=[
            jax.ShapeDtypeStruct((N, 128), jnp.float32),
            jax.ShapeDtypeStruct((N, 128), jnp.float32),
            jax.ShapeDtypeStruct((N, 8), jnp.float32),
        ],
    )(x, W_gat, Asd)


def _tcb_body(ea_ref, et_ref, elw_ref, tE_ref, aep_ref, tG_ref, epp_ref, ae8_ref):
    ea = ea_ref[...]
    et = et_ref[...]
    oh = (lax.broadcasted_iota(jnp.int32, (_BE, T), 1) == et).astype(jnp.float32)
    epp_ref[...] = (
        jnp.dot(ea, elw_ref[...], preferred_element_type=jnp.float32)
        + jnp.dot(oh, tE_ref[...], preferred_element_type=jnp.float32))
    ae8_ref[...] = (
        jnp.dot(ea, aep_ref[...], preferred_element_type=jnp.float32)
        + jnp.dot(oh, tG_ref[...], preferred_element_type=jnp.float32))


def _tcb(edge_attr, et2d, edge_lin_w, tE, AEP8, tG8):
    return pl.pallas_call(
        _tcb_body,
        grid=(E // _BE,),
        in_specs=[
            pl.BlockSpec((_BE, ED), lambda i: (i, 0)),
            pl.BlockSpec((_BE, 1), lambda i: (i, 0)),
            pl.BlockSpec((ED, D), lambda i: (0, 0)),
            pl.BlockSpec((T, D), lambda i: (0, 0)),
            pl.BlockSpec((ED, 8), lambda i: (0, 0)),
            pl.BlockSpec((T, 8), lambda i: (0, 0)),
        ],
        out_specs=[
            pl.BlockSpec((_BE, D), lambda i: (i, 0)),
            pl.BlockSpec((_BE, 8), lambda i: (i, 0)),
        ],
        out_shape=[
            jax.ShapeDtypeStruct((E, D), jnp.float32),
            jax.ShapeDtypeStruct((E, 8), jnp.float32),
        ],
    )(edge_attr, et2d, edge_lin_w, tE, AEP8, tG8)


def _zero_vmem_rows(ref, nrows, ncols):
    z16 = jnp.zeros((16,), jnp.float32)

    def body(r, _):
        for cb in range(ncols // 16):
            ref[r, pl.ds(cb * 16, 16)] = z16
        return 0

    lax.fori_loop(0, nrows, body, 0, unroll=False)


def _zero_my_shared_rows(zbuf, shared, base):
    for off, cnt in ZCHUNKS:
        pltpu.sync_copy(zbuf.at[pl.ds(0, cnt)], shared.at[pl.ds(base + off, cnt)])


def _dump_my_shared_rows(shared, out, cid, base):
    for off, cnt in ZCHUNKS:
        pltpu.sync_copy(shared.at[pl.ds(base + off, cnt)],
                        out.at[cid, pl.ds(base + off, cnt)])


def _scale_rows_by_heads(h_v, ex4_v, h_lo):
    def body(e, _):
        ef = jnp.full((16,), e * 4, jnp.int32)
        b0 = plsc.load_gather(ex4_v, [ef + h_lo])
        b1 = plsc.load_gather(ex4_v, [ef + (h_lo + 1)])
        for cb in range(8):
            b = b0 if cb < 4 else b1
            sl = pl.ds(cb * 16, 16)
            h_v[e, sl] = h_v[e, sl] * b
        return 0

    lax.fori_loop(0, EK, body, 0, unroll=False)


def _compute_ex(asrc_v, adst_v, ae_v, ex4_v, ex8_v, iota16):
    for j16 in range(8):
        rows = iota16 + (j16 * 16)
        for j in range(H):
            colj = jnp.full((16,), j, jnp.int32)
            a_s = plsc.load_gather(asrc_v, [rows, colj])
            a_d = plsc.load_gather(adst_v, [rows, colj + 4])
            a_e = plsc.load_gather(ae_v, [rows, colj])
            lg = a_s + a_d + a_e
            lg = jnp.maximum(lg, lg * 0.2)
            exv = jnp.exp(lg)
            plsc.store_scatter(ex4_v, [rows * 4 + j], exv)
            if ex8_v is not None:
                plsc.store_scatter(ex8_v, [rows, colj], exv)


def _sc_body(h0, h1, xH, asd, ae8, eppH, srcH, dstH,
             gat01_out, gat23_out, gine_out, denom_out,
             src_i, dst_i, ae_v, asrc_v, adst_v, ex4_v, ex8_v, h_v, epp_v,
             denom_sh, acc_sh):
    cid = lax.axis_index("c")
    sid = lax.axis_index("s")
    wid = sid * NC + cid
    base = sid * RPT

    iota16 = lax.iota(jnp.int32, 16)
    _zero_vmem_rows(h_v, 128, 128)

    def zex(g, _):
        rows = iota16 // 8 + 2 * g
        cols = jnp.bitwise_and(iota16, 7)
        plsc.store_scatter(ex8_v, [rows, cols], jnp.zeros((16,), jnp.float32))
        return 0

    lax.fori_loop(0, EK // 2, zex, 0, unroll=False)
    _zero_my_shared_rows(h_v, acc_sh, base)
    _zero_my_shared_rows(ex8_v, denom_sh, base)
    plsc.subcore_barrier()

    def chunk_a(i, _):
        c = wid + i * NW

        @pl.when(c < NCHUNK)
        def _():
            e0 = c * EK
            pltpu.sync_copy(srcH.at[pl.ds(e0, EK)], src_i)
            pltpu.sync_copy(dstH.at[pl.ds(e0, EK)], dst_i)
            pltpu.sync_copy(ae8.at[pl.ds(e0, EK)], ae_v)
            pltpu.sync_copy(asd.at[src_i], asrc_v)
            pltpu.sync_copy(asd.at[dst_i], adst_v)
            pltpu.sync_copy(h0.at[src_i], h_v)
            _compute_ex(asrc_v, adst_v, ae_v, ex4_v, ex8_v, iota16)
            _scale_rows_by_heads(h_v, ex4_v, 0)
            pltpu.sync_copy(ex8_v, denom_sh.at[dst_i], add=True)
            pltpu.sync_copy(h_v, acc_sh.at[dst_i], add=True)

        return 0

    lax.fori_loop(0, MAXCH, chunk_a, 0, unroll=False)
    plsc.subcore_barrier()
    _dump_my_shared_rows(acc_sh, gat01_out, cid, base)
    _dump_my_shared_rows(denom_sh, denom_out, cid, base)
    _zero_vmem_rows(h_v, 128, 128)
    _zero_my_shared_rows(h_v, acc_sh, base)
    plsc.subcore_barrier()

    def chunk_b(i, _):
        c = wid + i * NW

        @pl.when(c < NCHUNK)
        def _():
            e0 = c * EK
            pltpu.sync_copy(srcH.at[pl.ds(e0, EK)], src_i)
            pltpu.sync_copy(dstH.at[pl.ds(e0, EK)], dst_i)
            pltpu.sync_copy(ae8.at[pl.ds(e0, EK)], ae_v)
            pltpu.sync_copy(asd.at[src_i], asrc_v)
            pltpu.sync_copy(asd.at[dst_i], adst_v)
            pltpu.sync_copy(h1.at[src_i], h_v)
            _compute_ex(asrc_v, adst_v, ae_v, ex4_v, None, iota16)
            _scale_rows_by_heads(h_v, ex4_v, 2)
            pltpu.sync_copy(h_v, acc_sh.at[dst_i], add=True)

        return 0

    lax.fori_loop(0, MAXCH, chunk_b, 0, unroll=False)
    plsc.subcore_barrier()
    _dump_my_shared_rows(acc_sh, gat23_out, cid, base)
    _zero_vmem_rows(h_v, 128, 128)
    _zero_my_shared_rows(h_v, acc_sh, base)
    plsc.subcore_barrier()

    def chunk_c(i, _):
        c = wid + i * NW

        @pl.when(c < NCHUNK)
        def _():
            e0 = c * EK
            pltpu.sync_copy(srcH.at[pl.ds(e0, EK)], src_i)
            pltpu.sync_copy(dstH.at[pl.ds(e0, EK)], dst_i)
            pltpu.sync_copy(eppH.at[pl.ds(e0, EK)], epp_v)
            pltpu.sync_copy(xH.at[src_i], h_v)

            def body(e, _):
                for cb in range(8):
                    sl = pl.ds(cb * 16, 16)
                    v = h_v[e, sl] + epp_v[e, sl]
                    h_v[e, sl] = jnp.maximum(v, 0.0)
                return 0

            lax.fori_loop(0, EK, body, 0, unroll=False)
            pltpu.sync_copy(h_v, acc_sh.at[dst_i], add=True)

        return 0

    lax.fori_loop(0, MAXCH, chunk_c, 0, unroll=False)
    plsc.subcore_barrier()
    _dump_my_shared_rows(acc_sh, gine_out, cid, base)


def _sc(h0, h1, x, asd, ae8, epp, src, dst):
    return pl.kernel(
        _sc_body,
        out_type=[
            jax.ShapeDtypeStruct((NC, NP, 128), jnp.float32),
            jax.ShapeDtypeStruct((NC, NP, 128), jnp.float32),
            jax.ShapeDtypeStruct((NC, NP, 128), jnp.float32),
            jax.ShapeDtypeStruct((NC, NP, 8), jnp.float32),
        ],
        mesh=_mesh(),
        compiler_params=pltpu.CompilerParams(
            use_tc_tiling_on_sc=False, needs_layout_passes=False),
        scratch_types=[
            pltpu.VMEM((EK,), jnp.int32),
            pltpu.VMEM((EK,), jnp.int32),
            pltpu.VMEM((EK, 8), jnp.float32),
            pltpu.VMEM((EK, 8), jnp.float32),
            pltpu.VMEM((EK, 8), jnp.float32),
            pltpu.VMEM((EK * 4,), jnp.float32),
            pltpu.VMEM((EK, 8), jnp.float32),
            pltpu.VMEM((EK, 128), jnp.float32),
            pltpu.VMEM((EK, 128), jnp.float32),
            pltpu.VMEM_SHARED((NP, 8), jnp.float32),
            pltpu.VMEM_SHARED((NP, 128), jnp.float32),
        ],
    )(h0, h1, x, asd, ae8, epp, src, dst)


def _tcc_body(x_ref, g0_ref, g1_ref, d_ref, gi_ref, e2_ref, w1_ref, b1_ref,
              w2_ref, b2_ref, cwa0_ref, cwa1_ref, cwb_ref, zb_ref, lg_ref,
              lb_ref, out_ref):
    num0 = g0_ref[0] + g0_ref[1]
    num1 = g1_ref[0] + g1_ref[1]
    den = d_ref[0, :, :4] + d_ref[1, :, :4]
    dinv = 1.0 / (den + 1e-16)
    e2 = e2_ref[...]
    s01 = jnp.dot(dinv[:, :2], e2, preferred_element_type=jnp.float32)
    s23 = jnp.dot(dinv[:, 2:], e2, preferred_element_type=jnp.float32)
    z = (jnp.dot(num0 * s01, cwa0_ref[...], preferred_element_type=jnp.float32)
         + jnp.dot(num1 * s23, cwa1_ref[...], preferred_element_type=jnp.float32))
    hg = x_ref[...] + gi_ref[0] + gi_ref[1]
    t = jnp.maximum(
        jnp.dot(hg, w1_ref[...], preferred_element_type=jnp.float32)
        + b1_ref[...], 0.0)
    g = jnp.dot(t, w2_ref[...], preferred_element_type=jnp.float32) + b2_ref[...]
    z = z + jnp.dot(g, cwb_ref[...], preferred_element_type=jnp.float32) + zb_ref[...]
    mu = jnp.mean(z, axis=-1, keepdims=True)
    zc = z - mu
    var = jnp.mean(zc * zc, axis=-1, keepdims=True)
    zn = zc * lax.rsqrt(var + 1e-5) * lg_ref[...] + lb_ref[...]
    out_ref[...] = jnp.maximum(zn, 0.0)


def _tcc(x, gat0_p, gat1_p, denom_p, gine_p, E2, mlp_w1, mlp_b1, mlp_w2,
         mlp_b2, cwa0, cwa1, cwb, zb, ln_gamma, ln_beta):
    full = lambda *shape: pl.BlockSpec(shape, lambda i: (0,) * len(shape))
    return pl.pallas_call(
        _tcc_body,
        grid=(N // _BN,),
        in_specs=[
            pl.BlockSpec((_BN, D), lambda i: (i, 0)),
            pl.BlockSpec((NC, _BN, 128), lambda i: (0, i, 0)),
            pl.BlockSpec((NC, _BN, 128), lambda i: (0, i, 0)),
            pl.BlockSpec((NC, _BN, 8), lambda i: (0, i, 0)),
            pl.BlockSpec((NC, _BN, 128), lambda i: (0, i, 0)),
            full(2, 128),
            full(D, GINE),
            full(1, GINE),
            full(GINE, GINE),
            full(1, GINE),
            full(128, OUT),
            full(128, OUT),
            full(GINE, OUT),
            full(1, OUT),
            full(1, OUT),
            full(1, OUT),
        ],
        out_specs=pl.BlockSpec((_BN, OUT), lambda i: (i, 0)),
        out_shape=jax.ShapeDtypeStruct((N, OUT), jnp.float32),
    )(x, gat0_p, gat1_p, denom_p, gine_p, E2, mlp_w1, mlp_b1, mlp_w2, mlp_b2,
      cwa0, cwa1, cwb, zb, ln_gamma, ln_beta)


def kernel(x, edge_index, edge_attr, edge_types, type_emb_gat, W_gat,
           W_edge_gat, att_src, att_dst, att_edge, bias_gat, type_emb_gine,
           edge_lin_w, edge_lin_b, mlp_w1, mlp_b1, mlp_w2, mlp_b2, comb_w,
           comb_b, ln_gamma, ln_beta):
    src = edge_index[0].astype(jnp.int32)
    dst = edge_index[1].astype(jnp.int32)
    et2d = edge_types.astype(jnp.int32).reshape(E, 1)

    ar = jnp.arange(H)
    Asrc = jnp.zeros((H, C, H), jnp.float32).at[ar, :, ar].set(att_src)
    Adst = jnp.zeros((H, C, H), jnp.float32).at[ar, :, ar].set(att_dst)
    Asd = jnp.concatenate(
        [Asrc.reshape(H * C, H), Adst.reshape(H * C, H)], axis=1)
    AEP = jnp.einsum("ehc,hc->eh", W_edge_gat.reshape(ED, H, C), att_edge)
    AEP8 = jnp.pad(AEP, ((0, 0), (0, 4)))
    tG8 = jnp.dot(type_emb_gat, AEP8)
    tE = jnp.dot(type_emb_gine, edge_lin_w) + edge_lin_b[None]
    E2 = jnp.repeat(jnp.eye(2, dtype=jnp.float32), 64, axis=1)
    cwa0 = comb_w[:128]
    cwa1 = comb_w[128:256]
    cwb = comb_w[256:]
    zb = (comb_b + jnp.dot(bias_gat, comb_w[:256]))[None]

    h0, h1, asd = _tca(x, W_gat, Asd)
    epp, ae8 = _tcb(edge_attr, et2d, edge_lin_w, tE, AEP8, tG8)
    gat0_p, gat1_p, gine_p, denom_p = _sc(h0, h1, x, asd, ae8, epp, src, dst)
    return _tcc(x, gat0_p, gat1_p, denom_p, gine_p, E2, mlp_w1,
                mlp_b1.reshape(1, GINE), mlp_w2, mlp_b2.reshape(1, GINE),
                cwa0, cwa1, cwb, zb, ln_gamma.reshape(1, OUT),
                ln_beta.reshape(1, OUT))
